# Initial kernel scaffold; baseline (speedup 1.0000x reference)
import jax
import jax.numpy as jnp
from jax.experimental import pallas as pl


def kernel(h, coord, edge_index, coord_diff, edge_attr, W1, b1, W2, b2, W3):
    raise NotImplementedError("write your pallas kernel here")


# speedup vs baseline: 2.5920x; 2.5920x over previous
import functools

import jax
import jax.numpy as jnp
from jax import lax
from jax.experimental import pallas as pl
from jax.experimental.pallas import tpu as pltpu
from jax.experimental.pallas import tpu_sc as plsc

N = 10000
E = 320000
H = 128
NORM = 100.0

NW = 32
EW = E // NW
CS = 80
NCH = EW // CS


def _pre_body(h_ref, w1at_ref, w1bt_ref, pa_ref, pb_ref):
    hblk = h_ref[...]
    pa_ref[...] = jnp.dot(hblk, w1at_ref[...], preferred_element_type=jnp.float32)
    pb_ref[...] = jnp.dot(hblk, w1bt_ref[...], preferred_element_type=jnp.float32)


def _precompute(h, w1at, w1bt):
    bn = 400
    grid = (N // bn,)
    return pl.pallas_call(
        _pre_body,
        grid=grid,
        in_specs=[
            pl.BlockSpec((bn, H), lambda i: (i, 0)),
            pl.BlockSpec((H, H), lambda i: (0, 0)),
            pl.BlockSpec((H, H), lambda i: (0, 0)),
        ],
        out_specs=[
            pl.BlockSpec((bn, H), lambda i: (i, 0)),
            pl.BlockSpec((bn, H), lambda i: (i, 0)),
        ],
        out_shape=[
            jax.ShapeDtypeStruct((N, H), jnp.float32),
            jax.ShapeDtypeStruct((N, H), jnp.float32),
        ],
    )(h, w1at, w1bt)


def _gather_body(pa_hbm, pb_hbm, row_hbm, col_hbm, ga_hbm, gb_hbm,
                 idxa, idxb, bufa, bufb, sema, semb):
    c = lax.axis_index("c")
    s = lax.axis_index("s")
    wid = s * 2 + c
    base = wid * EW

    def step(j, carry):
        off = base + j * CS
        pltpu.sync_copy(row_hbm.at[pl.ds(off, CS)], idxa)
        pltpu.sync_copy(col_hbm.at[pl.ds(off, CS)], idxb)
        ca = pltpu.async_copy(pa_hbm.at[idxa], bufa, sema)
        cb = pltpu.async_copy(pb_hbm.at[idxb], bufb, semb)
        ca.wait()
        cb.wait()
        pltpu.sync_copy(bufa, ga_hbm.at[pl.ds(off, CS)])
        pltpu.sync_copy(bufb, gb_hbm.at[pl.ds(off, CS)])
        return carry

    lax.fori_loop(0, NCH, step, 0)


def _gather(pa, pb, row, col):
    mesh = plsc.VectorSubcoreMesh(core_axis_name="c", subcore_axis_name="s")
    k = pl.kernel(
        _gather_body,
        out_type=(
            jax.ShapeDtypeStruct((E, H), jnp.float32),
            jax.ShapeDtypeStruct((E, H), jnp.float32),
        ),
        mesh=mesh,
        scratch_types=[
            pltpu.VMEM((CS,), jnp.int32),
            pltpu.VMEM((CS,), jnp.int32),
            pltpu.VMEM((CS, H), jnp.float32),
            pltpu.VMEM((CS, H), jnp.float32),
            pltpu.SemaphoreType.DMA,
            pltpu.SemaphoreType.DMA,
        ],
    )
    return k(pa, pb, row, col)


def _mlp_body(ga_ref, gb_ref, ea_ref, w1c_ref, b1_ref, w2t_ref, b2_ref, w3_ref,
              phi_ref):
    x = ga_ref[...] + gb_ref[...] + ea_ref[...] * w1c_ref[...] + b1_ref[...]
    x = x * (1.0 / (1.0 + jnp.exp(-x)))
    x = jnp.dot(x, w2t_ref[...], preferred_element_type=jnp.float32) + b2_ref[...]
    x = x * (1.0 / (1.0 + jnp.exp(-x)))
    phi_ref[...] = jnp.sum(x * w3_ref[...], axis=1, keepdims=True)


def _edge_mlp(ga, gb, ea, w1c, b1r, w2t, b2r, w3r):
    bk = 2000
    grid = (E // bk,)
    wspec = pl.BlockSpec((1, H), lambda i: (0, 0))
    return pl.pallas_call(
        _mlp_body,
        grid=grid,
        in_specs=[
            pl.BlockSpec((bk, H), lambda i: (i, 0)),
            pl.BlockSpec((bk, H), lambda i: (i, 0)),
            pl.BlockSpec((bk, 1), lambda i: (i, 0)),
            wspec,
            wspec,
            pl.BlockSpec((H, H), lambda i: (0, 0)),
            wspec,
            wspec,
        ],
        out_specs=pl.BlockSpec((bk, 1), lambda i: (i, 0)),
        out_shape=jax.ShapeDtypeStruct((E, 1), jnp.float32),
    )(ga, gb, ea, w1c, b1r, w2t, b2r, w3r)


def _scatter_body(row_hbm, phi_hbm, cd_hbm, part_hbm,
                  idx_v, phi_v, cd_v, acc_v):
    c = lax.axis_index("c")
    s = lax.axis_index("s")
    wid = s * 2 + c
    base = wid * EW
    iota = lax.iota(jnp.int32, 16)

    def zero(i, carry):
        acc_v[pl.ds(i * 16, 16)] = jnp.zeros((16,), jnp.float32)
        return carry

    lax.fori_loop(0, 4 * N // 16, zero, 0)

    def chunk(j, carry):
        off = base + j * CS
        pltpu.sync_copy(row_hbm.at[pl.ds(off, CS)], idx_v)
        pltpu.sync_copy(phi_hbm.at[pl.ds(off, CS)], phi_v)
        pltpu.sync_copy(cd_hbm.at[pl.ds(3 * off, 3 * CS)], cd_v)

        def grp(g, carry2):
            l16 = g * 16 + iota
            p16 = phi_v[pl.ds(g * 16, 16)]
            e16 = idx_v[pl.ds(g * 16, 16)]
            for comp in range(3):
                cdc = plsc.load_gather(cd_v, [l16 * 3 + comp])
                plsc.addupdate_scatter(acc_v, [e16 * 4 + comp], p16 * cdc)
            return carry2

        lax.fori_loop(0, CS // 16, grp, 0)
        return carry

    lax.fori_loop(0, NCH, chunk, 0)
    pltpu.sync_copy(acc_v, part_hbm.at[wid])


def _scatter(row, phi, cdflat):
    mesh = plsc.VectorSubcoreMesh(core_axis_name="c", subcore_axis_name="s")
    k = pl.kernel(
        _scatter_body,
        out_type=jax.ShapeDtypeStruct((NW, 4 * N), jnp.float32),
        mesh=mesh,
        scratch_types=[
            pltpu.VMEM((CS,), jnp.int32),
            pltpu.VMEM((CS,), jnp.float32),
            pltpu.VMEM((3 * CS,), jnp.float32),
            pltpu.VMEM((4 * N,), jnp.float32),
        ],
        compiler_params=pltpu.CompilerParams(needs_layout_passes=False),
    )
    return k(row, phi, cdflat)


def _comb_body(part_ref, coord_ref, out_ref):
    agg = jnp.sum(part_ref[...], axis=0)
    out_ref[...] = coord_ref[...] + agg[:, :3] * (1.0 / NORM)


def _combine(parts, coord):
    bn = 400
    grid = (N // bn,)
    return pl.pallas_call(
        _comb_body,
        grid=grid,
        in_specs=[
            pl.BlockSpec((NW, bn, 4), lambda i: (0, i, 0)),
            pl.BlockSpec((bn, 3), lambda i: (i, 0)),
        ],
        out_specs=pl.BlockSpec((bn, 3), lambda i: (i, 0)),
        out_shape=jax.ShapeDtypeStruct((N, 3), jnp.float32),
    )(parts, coord)


def kernel(h, coord, edge_index, coord_diff, edge_attr, W1, b1, W2, b2, W3):
    row = edge_index[0]
    col = edge_index[1]
    w1at = W1[:, :H].T
    w1bt = W1[:, H:2 * H].T
    w1c = W1[:, 2 * H:].T
    b1r = b1.reshape(1, H)
    w2t = W2.T
    b2r = b2.reshape(1, H)
    w3r = W3
    pa, pb = _precompute(h, w1at, w1bt)
    ga, gb = _gather(pa, pb, row, col)
    phi = _edge_mlp(ga, gb, edge_attr, w1c, b1r, w2t, b2r, w3r).reshape(E)
    parts = _scatter(row, phi, coord_diff.reshape(-1)).reshape(NW, N, 4)
    return _combine(parts, coord)


# speedup vs baseline: 3.2013x; 1.2351x over previous
import functools

import jax
import jax.numpy as jnp
from jax import lax
from jax.experimental import pallas as pl
from jax.experimental.pallas import tpu as pltpu
from jax.experimental.pallas import tpu_sc as plsc

N = 10000
E = 320000
H = 128
HH = H // 2
NORM = 100.0

NW = 32
EW = E // NW

CW = 80
NCH = EW // CW
RING = 5

SCS = 2000
SNCH = EW // SCS


def _pack_bf16(lo_f32, hi_f32):
    ulo = lax.bitcast_convert_type(lo_f32, jnp.int32)
    uhi = lax.bitcast_convert_type(hi_f32, jnp.int32)
    lo = lax.shift_right_logical(ulo + 0x8000, 16)
    hi = (uhi + 0x8000) & jnp.int32(-65536)
    return lo | hi


def _unpack_bf16(packed_i32):
    lo = lax.bitcast_convert_type(lax.shift_left(packed_i32, 16), jnp.float32)
    hi = lax.bitcast_convert_type(packed_i32 & jnp.int32(-65536), jnp.float32)
    return lo, hi


def _pre_body(h_ref, w1at_ref, w1bt_ref, pa_ref, pb_ref):
    hblk = h_ref[...]
    pa_ref[...] = jnp.dot(hblk, w1at_ref[...], preferred_element_type=jnp.float32)
    pb_ref[...] = jnp.dot(hblk, w1bt_ref[...], preferred_element_type=jnp.float32)


def _precompute(h, w1at, w1bt):
    bn = 2000
    grid = (N // bn,)
    wspec = pl.BlockSpec((H, H), lambda i: (0, 0))
    return pl.pallas_call(
        _pre_body,
        grid=grid,
        in_specs=[pl.BlockSpec((bn, H), lambda i: (i, 0))] + [wspec] * 2,
        out_specs=[
            pl.BlockSpec((bn, H), lambda i: (i, 0)),
            pl.BlockSpec((bn, H), lambda i: (i, 0)),
        ],
        out_shape=[
            jax.ShapeDtypeStruct((N, H), jnp.float32),
            jax.ShapeDtypeStruct((N, H), jnp.float32),
        ],
    )(h, w1at, w1bt)


def _gather_body(pa_hbm, pb_hbm, row_hbm, col_hbm, ga_hbm, gb_hbm,
                 idxa, idxb, bufa, bufb, *sems):
    gsa = sems[0:RING]
    gsb = sems[RING:2 * RING]
    wsa = sems[2 * RING:3 * RING]
    wsb = sems[3 * RING:4 * RING]
    c = lax.axis_index("c")
    s = lax.axis_index("s")
    wid = s * 2 + c
    cbase = wid * NCH

    def do_fire(j, b):
        pltpu.sync_copy(row_hbm.at[cbase + j], idxa.at[b])
        pltpu.sync_copy(col_hbm.at[cbase + j], idxb.at[b])
        pltpu.async_copy(pa_hbm.at[idxa.at[b]], bufa.at[b], gsa[b])
        pltpu.async_copy(pb_hbm.at[idxb.at[b]], bufb.at[b], gsb[b])

    def do_writeout(j, b):
        pltpu.make_async_copy(pa_hbm.at[idxa.at[b]], bufa.at[b], gsa[b]).wait()
        pltpu.make_async_copy(pb_hbm.at[idxb.at[b]], bufb.at[b], gsb[b]).wait()
        off = (cbase + j) * CW
        pltpu.async_copy(bufa.at[b], ga_hbm.at[pl.ds(off, CW)], wsa[b])
        pltpu.async_copy(bufb.at[b], gb_hbm.at[pl.ds(off, CW)], wsb[b])

    def drain_writeout(b):
        pltpu.make_async_copy(bufa.at[b], ga_hbm.at[pl.ds(0, CW)], wsa[b]).wait()
        pltpu.make_async_copy(bufb.at[b], gb_hbm.at[pl.ds(0, CW)], wsb[b]).wait()

    def outer(go, carry):
        for b in range(RING):
            j = go * RING + b
            @pl.when(j >= 2)
            def _():
                do_writeout(j - 2, (b - 2) % RING)

            @pl.when(go >= 1)
            def _():
                drain_writeout(b)

            do_fire(j, b)
        return carry

    lax.fori_loop(0, NCH // RING, outer, 0)
    for j in (NCH - 2, NCH - 1):
        do_writeout(j, j % RING)
    for b in range(RING):
        drain_writeout(b)


def _gather(pa, pb, row2d, col2d):
    mesh = plsc.VectorSubcoreMesh(core_axis_name="c", subcore_axis_name="s")
    k = pl.kernel(
        _gather_body,
        out_type=(
            jax.ShapeDtypeStruct((E, H), jnp.float32),
            jax.ShapeDtypeStruct((E, H), jnp.float32),
        ),
        mesh=mesh,
        scratch_types=[
            pltpu.VMEM((RING, CW), jnp.int32),
            pltpu.VMEM((RING, CW), jnp.int32),
            pltpu.VMEM((RING, CW, H), jnp.float32),
            pltpu.VMEM((RING, CW, H), jnp.float32),
        ] + [pltpu.SemaphoreType.DMA] * (4 * RING),
    )
    return k(pa, pb, row2d, col2d)


def _mlp_body(ga_ref, gb_ref, ea_ref, w1c_ref, b1_ref, w2t_ref, b2_ref, w3_ref,
              phi_ref):
    x = ga_ref[...] + gb_ref[...] + ea_ref[...] * w1c_ref[...] + b1_ref[...]
    x = x * (1.0 / (1.0 + jnp.exp(-x)))
    x = jnp.dot(x.astype(jnp.bfloat16), w2t_ref[...],
                preferred_element_type=jnp.float32) + b2_ref[...]
    x = x * (1.0 / (1.0 + jnp.exp(-x)))
    phi_ref[...] = jnp.sum(x * w3_ref[...], axis=1, keepdims=True)


def _edge_mlp(ga, gb, ea, w1c, b1r, w2t, b2r, w3r):
    bk = 4000
    grid = (E // bk,)
    wspec = pl.BlockSpec((1, H), lambda i: (0, 0))
    return pl.pallas_call(
        _mlp_body,
        grid=grid,
        in_specs=[
            pl.BlockSpec((bk, H), lambda i: (i, 0)),
            pl.BlockSpec((bk, H), lambda i: (i, 0)),
            pl.BlockSpec((bk, 1), lambda i: (i, 0)),
            wspec,
            wspec,
            pl.BlockSpec((H, H), lambda i: (0, 0)),
            wspec,
            wspec,
        ],
        out_specs=pl.BlockSpec((bk, 1), lambda i: (i, 0)),
        out_shape=jax.ShapeDtypeStruct((E, 1), jnp.float32),
    )(ga, gb, ea, w1c, b1r, w2t, b2r, w3r)


def _scatter_body(row_hbm, phi_hbm, cd_hbm, part_hbm,
                  idx_v, phi_v, cd_v, acc_v):
    c = lax.axis_index("c")
    s = lax.axis_index("s")
    wid = s * 2 + c
    base = wid * EW
    iota = lax.iota(jnp.int32, 16)
    zeros16 = jnp.zeros((16,), jnp.float32)

    def zero(i, carry):
        for k in range(10):
            acc_v[pl.ds(i * 160 + k * 16, 16)] = zeros16
        return carry

    lax.fori_loop(0, 4 * N // 160, zero, 0)

    def chunk(j, carry):
        off = base + j * SCS
        pltpu.sync_copy(row_hbm.at[pl.ds(off, SCS)], idx_v)
        pltpu.sync_copy(phi_hbm.at[pl.ds(off, SCS)], phi_v)
        pltpu.sync_copy(cd_hbm.at[pl.ds(3 * off, 3 * SCS)], cd_v)

        def grp(g, carry2):
            l16 = g * 16 + iota
            p16 = phi_v[pl.ds(g * 16, 16)]
            e16 = idx_v[pl.ds(g * 16, 16)]
            for comp in range(3):
                cdc = plsc.load_gather(cd_v, [l16 * 3 + comp])
                plsc.addupdate_scatter(acc_v, [e16 * 4 + comp], p16 * cdc)
            return carry2

        lax.fori_loop(0, SCS // 16, grp, 0)
        return carry

    lax.fori_loop(0, SNCH, chunk, 0)
    pltpu.sync_copy(acc_v, part_hbm.at[wid])


def _scatter(row, phi, cdflat):
    mesh = plsc.VectorSubcoreMesh(core_axis_name="c", subcore_axis_name="s")
    k = pl.kernel(
        _scatter_body,
        out_type=jax.ShapeDtypeStruct((NW, 4 * N), jnp.float32),
        mesh=mesh,
        scratch_types=[
            pltpu.VMEM((SCS,), jnp.int32),
            pltpu.VMEM((SCS,), jnp.float32),
            pltpu.VMEM((3 * SCS,), jnp.float32),
            pltpu.VMEM((4 * N,), jnp.float32),
        ],
        compiler_params=pltpu.CompilerParams(needs_layout_passes=False),
    )
    return k(row, phi, cdflat)


def _comb_body(part_ref, coord_ref, out_ref):
    agg = jnp.sum(part_ref[...], axis=0)
    out_ref[...] = coord_ref[...] + agg[:, :3] * (1.0 / NORM)


def _combine(parts, coord):
    bn = 400
    grid = (N // bn,)
    return pl.pallas_call(
        _comb_body,
        grid=grid,
        in_specs=[
            pl.BlockSpec((NW, bn, 4), lambda i: (0, i, 0)),
            pl.BlockSpec((bn, 3), lambda i: (i, 0)),
        ],
        out_specs=pl.BlockSpec((bn, 3), lambda i: (i, 0)),
        out_shape=jax.ShapeDtypeStruct((N, 3), jnp.float32),
    )(parts, coord)


def kernel(h, coord, edge_index, coord_diff, edge_attr, W1, b1, W2, b2, W3):
    row = edge_index[0]
    col = edge_index[1]
    w1at = W1[:, :H].T
    w1bt = W1[:, H:2 * H].T
    w1c = W1[:, 2 * H:].T
    b1r = b1.reshape(1, H)
    w2t = W2.T.astype(jnp.bfloat16)
    b2r = b2.reshape(1, H)
    w3r = W3
    pa, pb = _precompute(h, w1at, w1bt)
    ga, gb = _gather(pa, pb, row.reshape(E // CW, CW), col.reshape(E // CW, CW))
    phi = _edge_mlp(ga, gb, edge_attr, w1c, b1r, w2t, b2r, w3r).reshape(E)
    parts = _scatter(row, phi, coord_diff.reshape(-1)).reshape(NW, N, 4)
    return _combine(parts, coord)


# speedup vs baseline: 5.3817x; 1.6811x over previous
import functools

import jax
import jax.numpy as jnp
from jax import lax
from jax.experimental import pallas as pl
from jax.experimental.pallas import tpu as pltpu
from jax.experimental.pallas import tpu_sc as plsc

N = 10000
E = 320000
H = 128
HH = H // 2
NORM = 100.0

NW = 32
EW = E // NW

CW = 80
NCH = EW // CW
RING = 5

SCS = 2000
SNCH = EW // SCS
NP = 10240


def _pack_bf16(lo_f32, hi_f32):
    ulo = lax.bitcast_convert_type(lo_f32, jnp.int32)
    uhi = lax.bitcast_convert_type(hi_f32, jnp.int32)
    lo = lax.shift_right_logical(ulo + 0x8000, 16)
    hi = (uhi + 0x8000) & jnp.int32(-65536)
    return lo | hi


def _unpack_bf16(packed_i32):
    lo = lax.bitcast_convert_type(lax.shift_left(packed_i32, 16), jnp.float32)
    hi = lax.bitcast_convert_type(packed_i32 & jnp.int32(-65536), jnp.float32)
    return lo, hi


def _pre_body(h_ref, w1at_ref, w1bt_ref, pa_ref, pb_ref):
    hblk = h_ref[...]
    pa_ref[...] = jnp.dot(hblk, w1at_ref[...], preferred_element_type=jnp.float32)
    pb_ref[...] = jnp.dot(hblk, w1bt_ref[...], preferred_element_type=jnp.float32)


def _precompute(h, w1at, w1bt):
    bn = 2000
    grid = (N // bn,)
    wspec = pl.BlockSpec((H, H), lambda i: (0, 0))
    return pl.pallas_call(
        _pre_body,
        grid=grid,
        in_specs=[pl.BlockSpec((bn, H), lambda i: (i, 0))] + [wspec] * 2,
        out_specs=[
            pl.BlockSpec((bn, H), lambda i: (i, 0)),
            pl.BlockSpec((bn, H), lambda i: (i, 0)),
        ],
        out_shape=[
            jax.ShapeDtypeStruct((N, H), jnp.float32),
            jax.ShapeDtypeStruct((N, H), jnp.float32),
        ],
    )(h, w1at, w1bt)


def _gather_body(pa_hbm, pb_hbm, row_hbm, col_hbm, ga_hbm, gb_hbm,
                 idxa, idxb, bufa, bufb, *sems):
    gsa = sems[0:RING]
    gsb = sems[RING:2 * RING]
    wsa = sems[2 * RING:3 * RING]
    wsb = sems[3 * RING:4 * RING]
    c = lax.axis_index("c")
    s = lax.axis_index("s")
    wid = s * 2 + c
    cbase = wid * NCH

    def do_fire(j, b):
        pltpu.sync_copy(row_hbm.at[cbase + j], idxa.at[b])
        pltpu.sync_copy(col_hbm.at[cbase + j], idxb.at[b])
        pltpu.async_copy(pa_hbm.at[idxa.at[b]], bufa.at[b], gsa[b])
        pltpu.async_copy(pb_hbm.at[idxb.at[b]], bufb.at[b], gsb[b])

    def do_writeout(j, b):
        pltpu.make_async_copy(pa_hbm.at[idxa.at[b]], bufa.at[b], gsa[b]).wait()
        pltpu.make_async_copy(pb_hbm.at[idxb.at[b]], bufb.at[b], gsb[b]).wait()
        off = (cbase + j) * CW
        pltpu.async_copy(bufa.at[b], ga_hbm.at[pl.ds(off, CW)], wsa[b])
        pltpu.async_copy(bufb.at[b], gb_hbm.at[pl.ds(off, CW)], wsb[b])

    def drain_writeout(b):
        pltpu.make_async_copy(bufa.at[b], ga_hbm.at[pl.ds(0, CW)], wsa[b]).wait()
        pltpu.make_async_copy(bufb.at[b], gb_hbm.at[pl.ds(0, CW)], wsb[b]).wait()

    def outer(go, carry):
        for b in range(RING):
            j = go * RING + b
            @pl.when(j >= 2)
            def _():
                do_writeout(j - 2, (b - 2) % RING)

            @pl.when(go >= 1)
            def _():
                drain_writeout(b)

            do_fire(j, b)
        return carry

    lax.fori_loop(0, NCH // RING, outer, 0)
    for j in (NCH - 2, NCH - 1):
        do_writeout(j, j % RING)
    for b in range(RING):
        drain_writeout(b)


def _gather(pa, pb, row2d, col2d):
    mesh = plsc.VectorSubcoreMesh(core_axis_name="c", subcore_axis_name="s")
    k = pl.kernel(
        _gather_body,
        out_type=(
            jax.ShapeDtypeStruct((E, H), jnp.float32),
            jax.ShapeDtypeStruct((E, H), jnp.float32),
        ),
        mesh=mesh,
        scratch_types=[
            pltpu.VMEM((RING, CW), jnp.int32),
            pltpu.VMEM((RING, CW), jnp.int32),
            pltpu.VMEM((RING, CW, H), jnp.float32),
            pltpu.VMEM((RING, CW, H), jnp.float32),
        ] + [pltpu.SemaphoreType.DMA] * (4 * RING),
    )
    return k(pa, pb, row2d, col2d)


def _mlp_body(ga_ref, gb_ref, ea_ref, w1c_ref, b1_ref, w2t_ref, b2_ref, w3_ref,
              phi_ref):
    x = ga_ref[...] + gb_ref[...] + ea_ref[...] * w1c_ref[...] + b1_ref[...]
    x = x * (1.0 / (1.0 + jnp.exp(-x)))
    x = jnp.dot(x.astype(jnp.bfloat16), w2t_ref[...],
                preferred_element_type=jnp.float32) + b2_ref[...]
    x = x * (1.0 / (1.0 + jnp.exp(-x)))
    phi = jnp.sum(jnp.reshape(x * w3_ref[...], (BKR, H, H)), axis=2)
    phi_ref[...] = jnp.reshape(phi, (1, BKR, H))


BK = 3200
BKR = BK // H


def _edge_mlp(ga, gb, ea, w1c, b1r, w2t, b2r, w3r):
    grid = (E // BK,)
    wspec = pl.BlockSpec((1, H), lambda i: (0, 0))
    return pl.pallas_call(
        _mlp_body,
        grid=grid,
        in_specs=[
            pl.BlockSpec((BK, H), lambda i: (i, 0)),
            pl.BlockSpec((BK, H), lambda i: (i, 0)),
            pl.BlockSpec((BK, 1), lambda i: (i, 0)),
            wspec,
            wspec,
            pl.BlockSpec((H, H), lambda i: (0, 0)),
            wspec,
            wspec,
        ],
        out_specs=pl.BlockSpec((1, BKR, H), lambda i: (i, 0, 0)),
        out_shape=jax.ShapeDtypeStruct((E // BK, BKR, H), jnp.float32),
    )(ga, gb, ea, w1c, b1r, w2t, b2r, w3r)


def _scatter_body(row_hbm, phi_hbm, cd0_hbm, cd1_hbm, cd2_hbm, part_hbm,
                  idx_v, phi_v, cd0_v, cd1_v, cd2_v, acc_v):
    c = lax.axis_index("c")
    s = lax.axis_index("s")
    wid = s * 2 + c
    base = wid * EW
    iota = lax.iota(jnp.int32, 16)
    zeros16 = jnp.zeros((16,), jnp.float32)

    def zero(i, carry):
        for k in range(10):
            acc_v[pl.ds(i * 160 + k * 16, 16)] = zeros16
        return carry

    lax.fori_loop(0, 3 * NP // 160, zero, 0)

    def chunk(j, carry):
        off = base + j * SCS
        pltpu.sync_copy(row_hbm.at[pl.ds(off, SCS)], idx_v)
        pltpu.sync_copy(phi_hbm.at[pl.ds(off, SCS)], phi_v)
        pltpu.sync_copy(cd0_hbm.at[pl.ds(off, SCS)], cd0_v)
        pltpu.sync_copy(cd1_hbm.at[pl.ds(off, SCS)], cd1_v)
        pltpu.sync_copy(cd2_hbm.at[pl.ds(off, SCS)], cd2_v)

        def grp(g, carry2):
            p16 = phi_v[pl.ds(g * 16, 16)]
            e16 = idx_v[pl.ds(g * 16, 16)]
            for comp, cdv in enumerate((cd0_v, cd1_v, cd2_v)):
                cdc = cdv[pl.ds(g * 16, 16)]
                plsc.addupdate_scatter(acc_v, [e16 + comp * NP], p16 * cdc)
            return carry2

        lax.fori_loop(0, SCS // 16, grp, 0)
        return carry

    lax.fori_loop(0, SNCH, chunk, 0)
    pltpu.sync_copy(acc_v, part_hbm.at[wid])


def _scatter(row, phi, cd0, cd1, cd2):
    mesh = plsc.VectorSubcoreMesh(core_axis_name="c", subcore_axis_name="s")
    k = pl.kernel(
        _scatter_body,
        out_type=jax.ShapeDtypeStruct((NW, 3 * NP), jnp.float32),
        mesh=mesh,
        scratch_types=[
            pltpu.VMEM((SCS,), jnp.int32),
            pltpu.VMEM((SCS,), jnp.float32),
            pltpu.VMEM((SCS,), jnp.float32),
            pltpu.VMEM((SCS,), jnp.float32),
            pltpu.VMEM((SCS,), jnp.float32),
            pltpu.VMEM((3 * NP,), jnp.float32),
        ],
        compiler_params=pltpu.CompilerParams(needs_layout_passes=False),
    )
    return k(row, phi, cd0, cd1, cd2)


def _comb_body(part_ref, coordt_ref, out_ref):
    p = part_ref[...]
    planes = [jnp.sum(p[:, comp * NP:(comp + 1) * NP], axis=0)[:N]
              for comp in range(3)]
    out_ref[...] = coordt_ref[...] + jnp.stack(planes, axis=0) * (1.0 / NORM)


def _combine(parts, coordt):
    return pl.pallas_call(
        _comb_body,
        grid=(1,),
        in_specs=[
            pl.BlockSpec((NW, 3 * NP), lambda i: (0, 0)),
            pl.BlockSpec((3, N), lambda i: (0, 0)),
        ],
        out_specs=pl.BlockSpec((3, N), lambda i: (0, 0)),
        out_shape=jax.ShapeDtypeStruct((3, N), jnp.float32),
    )(parts, coordt)


def kernel(h, coord, edge_index, coord_diff, edge_attr, W1, b1, W2, b2, W3):
    row = edge_index[0]
    col = edge_index[1]
    w1at = W1[:, :H].T
    w1bt = W1[:, H:2 * H].T
    w1c = W1[:, 2 * H:].T
    b1r = b1.reshape(1, H)
    w2t = W2.T.astype(jnp.bfloat16)
    b2r = b2.reshape(1, H)
    w3r = W3
    pa, pb = _precompute(h, w1at, w1bt)
    ga, gb = _gather(pa, pb, row.reshape(E // CW, CW), col.reshape(E // CW, CW))
    phi = _edge_mlp(ga, gb, edge_attr, w1c, b1r, w2t, b2r, w3r).reshape(E)
    cdt = coord_diff.T
    parts = _scatter(row, phi, cdt[0], cdt[1], cdt[2])
    return _combine(parts, coord.T).T


# speedup vs baseline: 6.7557x; 1.2553x over previous
import functools

import jax
import jax.numpy as jnp
from jax import lax
from jax.experimental import pallas as pl
from jax.experimental.pallas import tpu as pltpu
from jax.experimental.pallas import tpu_sc as plsc

N = 10000
E = 320000
H = 128
HH = H // 2
NORM = 100.0

NW = 32
EW = E // NW

CW = 80
RING = 5
NSL = 5
ES = E // NSL
NCH = ES // (NW * CW)

SCS = 2000
SNCH = EW // SCS
NP = 10240


def _pack_bf16(lo_f32, hi_f32):
    ulo = lax.bitcast_convert_type(lo_f32, jnp.int32)
    uhi = lax.bitcast_convert_type(hi_f32, jnp.int32)
    lo = lax.shift_right_logical(ulo + 0x8000, 16)
    hi = (uhi + 0x8000) & jnp.int32(-65536)
    return lo | hi


def _unpack_bf16(packed_i32):
    lo = lax.bitcast_convert_type(lax.shift_left(packed_i32, 16), jnp.float32)
    hi = lax.bitcast_convert_type(packed_i32 & jnp.int32(-65536), jnp.float32)
    return lo, hi


def _pre_body(h_ref, w1at_ref, w1bt_ref, pa_ref, pb_ref):
    hblk = h_ref[...]
    pa_ref[...] = jnp.dot(hblk, w1at_ref[...], preferred_element_type=jnp.float32)
    pb_ref[...] = jnp.dot(hblk, w1bt_ref[...], preferred_element_type=jnp.float32)


def _precompute(h, w1at, w1bt):
    bn = 2000
    grid = (N // bn,)
    wspec = pl.BlockSpec((H, H), lambda i: (0, 0))
    return pl.pallas_call(
        _pre_body,
        grid=grid,
        in_specs=[pl.BlockSpec((bn, H), lambda i: (i, 0))] + [wspec] * 2,
        out_specs=[
            pl.BlockSpec((bn, H), lambda i: (i, 0)),
            pl.BlockSpec((bn, H), lambda i: (i, 0)),
        ],
        out_shape=[
            jax.ShapeDtypeStruct((N, H), jnp.float32),
            jax.ShapeDtypeStruct((N, H), jnp.float32),
        ],
    )(h, w1at, w1bt)


def _gather_body(pa_hbm, pb_hbm, row_hbm, col_hbm, ga_hbm, gb_hbm,
                 idxa, idxb, bufa, bufb, *sems):
    gsa = sems[0:RING]
    gsb = sems[RING:2 * RING]
    wsa = sems[2 * RING:3 * RING]
    wsb = sems[3 * RING:4 * RING]
    c = lax.axis_index("c")
    s = lax.axis_index("s")
    wid = s * 2 + c
    cbase = wid * NCH

    def do_fire(j, b):
        pltpu.sync_copy(row_hbm.at[cbase + j], idxa.at[b])
        pltpu.sync_copy(col_hbm.at[cbase + j], idxb.at[b])
        pltpu.async_copy(pa_hbm.at[idxa.at[b]], bufa.at[b], gsa[b])
        pltpu.async_copy(pb_hbm.at[idxb.at[b]], bufb.at[b], gsb[b])

    def do_writeout(j, b):
        pltpu.make_async_copy(pa_hbm.at[idxa.at[b]], bufa.at[b], gsa[b]).wait()
        pltpu.make_async_copy(pb_hbm.at[idxb.at[b]], bufb.at[b], gsb[b]).wait()
        off = (cbase + j) * CW
        pltpu.async_copy(bufa.at[b], ga_hbm.at[pl.ds(off, CW)], wsa[b])
        pltpu.async_copy(bufb.at[b], gb_hbm.at[pl.ds(off, CW)], wsb[b])

    def drain_writeout(b):
        pltpu.make_async_copy(bufa.at[b], ga_hbm.at[pl.ds(0, CW)], wsa[b]).wait()
        pltpu.make_async_copy(bufb.at[b], gb_hbm.at[pl.ds(0, CW)], wsb[b]).wait()

    def outer(go, carry):
        for b in range(RING):
            j = go * RING + b
            @pl.when(j >= 2)
            def _():
                do_writeout(j - 2, (b - 2) % RING)

            @pl.when(go >= 1)
            def _():
                drain_writeout(b)

            do_fire(j, b)
        return carry

    lax.fori_loop(0, NCH // RING, outer, 0)
    for j in (NCH - 2, NCH - 1):
        do_writeout(j, j % RING)
    for b in range(RING):
        drain_writeout(b)


def _gather(pa, pb, row2d, col2d):
    mesh = plsc.VectorSubcoreMesh(core_axis_name="c", subcore_axis_name="s")
    k = pl.kernel(
        _gather_body,
        out_type=(
            jax.ShapeDtypeStruct((ES, H), jnp.float32),
            jax.ShapeDtypeStruct((ES, H), jnp.float32),
        ),
        mesh=mesh,
        scratch_types=[
            pltpu.VMEM((RING, CW), jnp.int32),
            pltpu.VMEM((RING, CW), jnp.int32),
            pltpu.VMEM((RING, CW, H), jnp.float32),
            pltpu.VMEM((RING, CW, H), jnp.float32),
        ] + [pltpu.SemaphoreType.DMA] * (4 * RING),
    )
    return k(pa, pb, row2d, col2d)


def _mlp_body(ga_ref, gb_ref, ea_ref, w1c_ref, b1_ref, w2t_ref, b2_ref, w3_ref,
              phi_ref):
    x = ga_ref[...] + gb_ref[...] + b1_ref[...]
    ea3 = ea_ref[...][0][:, :, None]
    x = x + jnp.reshape(ea3 * jnp.reshape(w1c_ref[...], (1, 1, H)), (BK, H))
    x = x * (1.0 / (1.0 + jnp.exp(-x)))
    x = jnp.dot(x.astype(jnp.bfloat16), w2t_ref[...],
                preferred_element_type=jnp.float32) + b2_ref[...]
    x = x * (1.0 / (1.0 + jnp.exp(-x)))
    phi = jnp.sum(jnp.reshape(x * w3_ref[...], (BKR, H, H)), axis=2)
    phi_ref[...] = jnp.reshape(phi, (1, BKR, H))


BK = 3200
BKR = BK // H


def _edge_mlp(ga, gb, ea2d, w1c, b1r, w2t, b2r, w3r):
    grid = (ES // BK,)
    wspec = pl.BlockSpec((1, H), lambda i: (0, 0))
    return pl.pallas_call(
        _mlp_body,
        grid=grid,
        in_specs=[
            pl.BlockSpec((BK, H), lambda i: (i, 0)),
            pl.BlockSpec((BK, H), lambda i: (i, 0)),
            pl.BlockSpec((1, BKR, H), lambda i: (i, 0, 0)),
            wspec,
            wspec,
            pl.BlockSpec((H, H), lambda i: (0, 0)),
            wspec,
            wspec,
        ],
        out_specs=pl.BlockSpec((1, BKR, H), lambda i: (i, 0, 0)),
        out_shape=jax.ShapeDtypeStruct((ES // BK, BKR, H), jnp.float32),
    )(ga, gb, ea2d.reshape(ES // BK, BKR, H), w1c, b1r, w2t, b2r, w3r)


def _scatter_body(row_hbm, phi_hbm, cd0_hbm, cd1_hbm, cd2_hbm, part_hbm,
                  idx_v, phi_v, cd0_v, cd1_v, cd2_v, acc_v):
    c = lax.axis_index("c")
    s = lax.axis_index("s")
    wid = s * 2 + c
    base = wid * EW
    iota = lax.iota(jnp.int32, 16)
    zeros16 = jnp.zeros((16,), jnp.float32)

    def zero(i, carry):
        for k in range(10):
            acc_v[pl.ds(i * 160 + k * 16, 16)] = zeros16
        return carry

    lax.fori_loop(0, 3 * NP // 160, zero, 0)

    def chunk(j, carry):
        off = base + j * SCS
        pltpu.sync_copy(row_hbm.at[pl.ds(off, SCS)], idx_v)
        pltpu.sync_copy(phi_hbm.at[pl.ds(off, SCS)], phi_v)
        pltpu.sync_copy(cd0_hbm.at[pl.ds(off, SCS)], cd0_v)
        pltpu.sync_copy(cd1_hbm.at[pl.ds(off, SCS)], cd1_v)
        pltpu.sync_copy(cd2_hbm.at[pl.ds(off, SCS)], cd2_v)

        def grp(g, carry2):
            p16 = phi_v[pl.ds(g * 16, 16)]
            e16 = idx_v[pl.ds(g * 16, 16)]
            for comp, cdv in enumerate((cd0_v, cd1_v, cd2_v)):
                cdc = cdv[pl.ds(g * 16, 16)]
                plsc.addupdate_scatter(acc_v, [e16 + comp * NP], p16 * cdc)
            return carry2

        lax.fori_loop(0, SCS // 16, grp, 0)
        return carry

    lax.fori_loop(0, SNCH, chunk, 0)
    pltpu.sync_copy(acc_v, part_hbm.at[wid])


def _scatter(row, phi, cd0, cd1, cd2):
    mesh = plsc.VectorSubcoreMesh(core_axis_name="c", subcore_axis_name="s")
    k = pl.kernel(
        _scatter_body,
        out_type=jax.ShapeDtypeStruct((NW, 3 * NP), jnp.float32),
        mesh=mesh,
        scratch_types=[
            pltpu.VMEM((SCS,), jnp.int32),
            pltpu.VMEM((SCS,), jnp.float32),
            pltpu.VMEM((SCS,), jnp.float32),
            pltpu.VMEM((SCS,), jnp.float32),
            pltpu.VMEM((SCS,), jnp.float32),
            pltpu.VMEM((3 * NP,), jnp.float32),
        ],
        compiler_params=pltpu.CompilerParams(needs_layout_passes=False),
    )
    return k(row, phi, cd0, cd1, cd2)


def _comb_body(part_ref, coordt_ref, out_ref):
    p = part_ref[...]
    planes = [jnp.sum(p[:, comp * NP:(comp + 1) * NP], axis=0)[:N]
              for comp in range(3)]
    out_ref[...] = coordt_ref[...] + jnp.stack(planes, axis=0) * (1.0 / NORM)


def _combine(parts, coordt):
    return pl.pallas_call(
        _comb_body,
        grid=(1,),
        in_specs=[
            pl.BlockSpec((NW, 3 * NP), lambda i: (0, 0)),
            pl.BlockSpec((3, N), lambda i: (0, 0)),
        ],
        out_specs=pl.BlockSpec((3, N), lambda i: (0, 0)),
        out_shape=jax.ShapeDtypeStruct((3, N), jnp.float32),
    )(parts, coordt)


def kernel(h, coord, edge_index, coord_diff, edge_attr, W1, b1, W2, b2, W3):
    row = edge_index[0]
    col = edge_index[1]
    w1at = W1[:, :H].T
    w1bt = W1[:, H:2 * H].T
    w1c = W1[:, 2 * H:].T
    b1r = b1.reshape(1, H)
    w2t = W2.T.astype(jnp.bfloat16)
    b2r = b2.reshape(1, H)
    w3r = W3
    pa, pb = _precompute(h, w1at, w1bt)
    row2d = row.reshape(E // CW, CW)
    col2d = col.reshape(E // CW, CW)
    ea2d = edge_attr.reshape(E // H, H)
    scw = ES // CW
    sea = ES // H
    phis = []
    for s in range(NSL):
        ga, gb = _gather(pa, pb,
                         row2d[s * scw:(s + 1) * scw],
                         col2d[s * scw:(s + 1) * scw])
        phis.append(_edge_mlp(ga, gb, ea2d[s * sea:(s + 1) * sea],
                              w1c, b1r, w2t, b2r, w3r))
    phi = jnp.concatenate(phis, axis=0).reshape(E)
    cdt = coord_diff.T
    parts = _scatter(row, phi, cdt[0], cdt[1], cdt[2])
    return _combine(parts, coord.T).T
